# Initial kernel scaffold; baseline (speedup 1.0000x reference)
import jax
import jax.numpy as jnp
from jax.experimental import pallas as pl


def kernel(x, edge_index, W1, b1, a1, W2, b2):
    raise NotImplementedError("write your pallas kernel here")


# speedup vs baseline: 9.6253x; 9.6253x over previous
import functools

import jax
import jax.numpy as jnp
from jax import lax
from jax.experimental import pallas as pl
from jax.experimental.pallas import tpu as pltpu
from jax.experimental.pallas import tpu_sc as plsc

N = 10000
D = 128
E = 320000

NC = 2
NS = 16
L = 16

NP = 10240
ROWS_PER_TILE = NP // NS
CH = 128
TRASH = N

E_TOT = E + N
CPT_E = -(-E_TOT // (NS * CH))
E_PAD = CPT_E * NS * CH
CPT_D = -(-E // (NC * NS * CH))
E_DPAD = CPT_D * NC * NS * CH

@functools.lru_cache(maxsize=None)
def _sc_mesh():
    return plsc.VectorSubcoreMesh(
        core_axis_name="c", subcore_axis_name="s",
        num_cores=NC, num_subcores=NS)


def _deg_body(src_hbm, dst_hbm, perm_hbm, deg_out, q_out,
              perm_v, ones_v, tmp_v, srcb, dstb, qb, deg_sh):
    c = lax.axis_index("c")
    s = lax.axis_index("s")
    w = c * NS + s

    def _fill_ones(r, _):
        ones_v[r, :] = jnp.ones((L,), jnp.float32)
        tmp_v[r, :] = jnp.zeros((L,), jnp.float32)
        return 0
    lax.fori_loop(0, CH, _fill_ones, 0)

    def _zero_slice(m, _):
        pltpu.sync_copy(tmp_v, deg_sh.at[pl.ds(s * ROWS_PER_TILE + m * CH, CH)])
        return 0
    lax.fori_loop(0, ROWS_PER_TILE // CH, _zero_slice, 0)
    pltpu.sync_copy(perm_hbm, perm_v)
    plsc.subcore_barrier()

    base = w * CPT_D * CH

    def _chunk(j, _):
        eb = base + j * CH
        pltpu.sync_copy(src_hbm.at[pl.ds(eb, CH)], srcb)
        pltpu.sync_copy(dst_hbm.at[pl.ds(eb, CH)], dstb)
        pltpu.sync_copy(ones_v, deg_sh.at[dstb], add=True)

        def _qgather(k, _2):
            iv = srcb[pl.ds(k * L, L)]
            qb[pl.ds(k * L, L)] = plsc.load_gather(perm_v, [iv])
            return 0
        lax.fori_loop(0, CH // L, _qgather, 0)
        pltpu.sync_copy(qb, q_out.at[pl.ds(eb, CH)])
        return 0
    lax.fori_loop(0, CPT_D, _chunk, 0)
    plsc.subcore_barrier()

    def _wb(m, _):
        rb = s * ROWS_PER_TILE + m * CH
        pltpu.sync_copy(deg_sh.at[pl.ds(rb, CH)], tmp_v)
        pltpu.sync_copy(tmp_v, deg_out.at[c, pl.ds(rb, CH)])
        return 0
    lax.fori_loop(0, ROWS_PER_TILE // CH, _wb, 0)


@functools.lru_cache(maxsize=None)
def _deg_call():
  return pl.kernel(
    _deg_body,
    out_type=(
        jax.ShapeDtypeStruct((NC, NP, L), jnp.float32),
        jax.ShapeDtypeStruct((E_DPAD,), jnp.int32),
    ),
    mesh=_sc_mesh(),
    compiler_params=pltpu.CompilerParams(needs_layout_passes=False),
    scratch_types=[
        pltpu.VMEM((NP,), jnp.int32),
        pltpu.VMEM((CH, L), jnp.float32),
        pltpu.VMEM((CH, L), jnp.float32),
        pltpu.VMEM((CH,), jnp.int32),
        pltpu.VMEM((CH,), jnp.int32),
        pltpu.VMEM((CH,), jnp.int32),
        pltpu.VMEM_SHARED((NP, L), jnp.float32),
    ],
  )


def _conv_body(tt_hbm, g_hbm, dst_hbm, out_hbm, rows0, gb, db, accum_sh):
    c = lax.axis_index("c")
    s = lax.axis_index("s")

    def _zero_row(r, _):
        def _zc(k, _2):
            rows0[r, pl.ds(k * L, L)] = jnp.zeros((L,), jnp.float32)
            return 0
        lax.fori_loop(0, D // L, _zc, 0)
        return 0
    lax.fori_loop(0, CH, _zero_row, 0)

    def _zero_slice(m, _):
        pltpu.sync_copy(rows0, accum_sh.at[pl.ds(s * ROWS_PER_TILE + m * CH, CH)])
        return 0
    lax.fori_loop(0, ROWS_PER_TILE // CH, _zero_slice, 0)
    plsc.subcore_barrier()

    base = s * CPT_E * CH

    def _chunk(j, _):
        eb = base + j * CH
        pltpu.sync_copy(g_hbm.at[c, pl.ds(eb, CH)], gb)
        pltpu.sync_copy(dst_hbm.at[pl.ds(eb, CH)], db)
        pltpu.sync_copy(tt_hbm.at[gb], rows0)
        pltpu.sync_copy(rows0, accum_sh.at[db], add=True)
        return 0
    lax.fori_loop(0, CPT_E, _chunk, 0)
    plsc.subcore_barrier()

    def _wb(m, _):
        rb = s * ROWS_PER_TILE + m * CH
        pltpu.sync_copy(accum_sh.at[pl.ds(rb, CH)], rows0)
        pltpu.sync_copy(rows0, out_hbm.at[c, pl.ds(rb, CH)])
        return 0
    lax.fori_loop(0, ROWS_PER_TILE // CH, _wb, 0)


@functools.lru_cache(maxsize=None)
def _conv_call():
  return pl.kernel(
    _conv_body,
    out_type=jax.ShapeDtypeStruct((NC, NP, D), jnp.float32),
    mesh=_sc_mesh(),
    scratch_types=[
        pltpu.VMEM((CH, D), jnp.float32),
        pltpu.VMEM((CH,), jnp.int32),
        pltpu.VMEM((CH,), jnp.int32),
        pltpu.VMEM_SHARED((NP, D), jnp.float32),
    ],
  )


_BM = 512


def _tc1_body(x_ref, w_ref, dinv_ref, dperm_ref, out_ref):
    h = jnp.dot(x_ref[...], w_ref[...], preferred_element_type=jnp.float32)
    out_ref[0] = h * dinv_ref[...]
    out_ref[1] = h * dperm_ref[...]


def _tc1_call(x_pad, w1, dinv_c, dperm_c):
    grid = NP // _BM
    return pl.pallas_call(
        _tc1_body,
        grid=(grid,),
        in_specs=[
            pl.BlockSpec((_BM, D), lambda i: (i, 0)),
            pl.BlockSpec((D, D), lambda i: (0, 0)),
            pl.BlockSpec((_BM, 1), lambda i: (i, 0)),
            pl.BlockSpec((_BM, 1), lambda i: (i, 0)),
        ],
        out_specs=pl.BlockSpec((2, _BM, D), lambda i: (0, i, 0)),
        out_shape=jax.ShapeDtypeStruct((2, NP, D), jnp.float32),
    )(x_pad, w1, dinv_c, dperm_c)


def _tc2_body(a_ref, w_ref, dinv_ref, b_ref, alpha_ref, out_ref):
    dinv = dinv_ref[...]
    alpha = alpha_ref[0, 0]
    p = a_ref[0] * dinv + b_ref[...]
    p = jnp.where(p > 0, p, alpha * p)
    q = a_ref[1] * dinv + b_ref[...]
    q = jnp.where(q > 0, q, alpha * q)
    out_ref[0] = jnp.dot(p, w_ref[...], preferred_element_type=jnp.float32) * dinv
    out_ref[1] = jnp.dot(q, w_ref[...], preferred_element_type=jnp.float32) * dinv


def _tc2_call(a1_, w2, dinv_c, b1_, alpha_):
    grid = NP // _BM
    return pl.pallas_call(
        _tc2_body,
        grid=(grid,),
        in_specs=[
            pl.BlockSpec((2, _BM, D), lambda i: (0, i, 0)),
            pl.BlockSpec((D, D), lambda i: (0, 0)),
            pl.BlockSpec((_BM, 1), lambda i: (i, 0)),
            pl.BlockSpec((1, D), lambda i: (0, 0)),
            pl.BlockSpec((1, 1), lambda i: (0, 0)),
        ],
        out_specs=pl.BlockSpec((2, _BM, D), lambda i: (0, i, 0)),
        out_shape=jax.ShapeDtypeStruct((2, NP, D), jnp.float32),
    )(a1_, w2, dinv_c, b1_, alpha_)


_BM3 = 80
_NBLK3 = N // _BM3


def _tc3_body(a_ref, dinv_ref, b_ref, pos_ref, neg_ref, sum_ref):
    i = pl.program_id(0)
    dinv = dinv_ref[...]
    b = b_ref[...]
    p = a_ref[0] * dinv + b
    q = a_ref[1] * dinv + b
    pos_ref[...] = p
    neg_ref[...] = q
    colsum = jnp.sum(p, axis=0, keepdims=True)
    prev = jnp.where(i == 0, jnp.zeros_like(colsum), sum_ref[...])
    acc = prev + colsum
    sum_ref[...] = jnp.where(i == _NBLK3 - 1,
                             jax.nn.sigmoid(acc * (1.0 / N)), acc)


def _tc3_call(a2_, dinv_c, b2_):
    return pl.pallas_call(
        _tc3_body,
        grid=(_NBLK3,),
        in_specs=[
            pl.BlockSpec((2, _BM3, D), lambda i: (0, i, 0)),
            pl.BlockSpec((_BM3, 1), lambda i: (i, 0)),
            pl.BlockSpec((1, D), lambda i: (0, 0)),
        ],
        out_specs=[
            pl.BlockSpec((_BM3, D), lambda i: (i, 0)),
            pl.BlockSpec((_BM3, D), lambda i: (i, 0)),
            pl.BlockSpec((1, D), lambda i: (0, 0)),
        ],
        out_shape=[
            jax.ShapeDtypeStruct((N, D), jnp.float32),
            jax.ShapeDtypeStruct((N, D), jnp.float32),
            jax.ShapeDtypeStruct((1, D), jnp.float32),
        ],
    )(a2_, dinv_c, b2_)


def kernel(x, edge_index, W1, b1, a1, W2, b2):
    src = edge_index[0].astype(jnp.int32)
    dst = edge_index[1].astype(jnp.int32)
    perm = jax.random.permutation(jax.random.key(42), N).astype(jnp.int32)
    perm_inv = jnp.argsort(perm).astype(jnp.int32)
    iota = jnp.arange(N, dtype=jnp.int32)

    pad_d = E_DPAD - E
    srcp = jnp.concatenate([src, jnp.zeros((pad_d,), jnp.int32)])
    dstp = jnp.concatenate([dst, jnp.full((pad_d,), TRASH, jnp.int32)])
    perm_pad = jnp.concatenate([perm, jnp.zeros((NP - N,), jnp.int32)])
    deg2, q = _deg_call()(srcp, dstp, perm_pad)
    deg = deg2[0, :, 0] + deg2[1, :, 0] + 1.0
    dinv = lax.rsqrt(deg)
    dperm = jnp.concatenate([dinv[perm_inv], jnp.ones((NP - N,), jnp.float32)])
    dinv_c = dinv[:, None]
    dperm_c = dperm[:, None]

    x_pad = jnp.concatenate([x, jnp.zeros((NP - N, D), x.dtype)])
    tt1 = _tc1_call(x_pad, W1, dinv_c, dperm_c)

    pad_e = E_PAD - E_TOT
    zpad = jnp.zeros((pad_e,), jnp.int32)
    g_pos = jnp.concatenate([src, iota, zpad])
    g_neg = jnp.concatenate([q[:E], perm, zpad]) + NP
    g1 = jnp.stack([g_pos, g_neg])
    g2 = jnp.stack([g_pos, g_pos + NP])
    dste = jnp.concatenate([dst, iota, jnp.full((pad_e,), TRASH, jnp.int32)])

    a1_acc = _conv_call()(tt1.reshape(2 * NP, D), g1, dste)

    tt2 = _tc2_call(a1_acc, W2, dinv_c, b1.reshape(1, D), a1.reshape(1, 1))

    a2_acc = _conv_call()(tt2.reshape(2 * NP, D), g2, dste)

    pos, neg, summ = _tc3_call(a2_acc, dinv_c, b2.reshape(1, D))
    return (pos, neg, summ.reshape(D))


# speedup vs baseline: 14.5134x; 1.5078x over previous
import functools

import jax
import jax.numpy as jnp
from jax import lax
from jax.experimental import pallas as pl
from jax.experimental.pallas import tpu as pltpu
from jax.experimental.pallas import tpu_sc as plsc

N = 10000
D = 128
E = 320000

NC = 2
NS = 16
L = 16

NP = 10240
ROWS_PER_TILE = NP // NS
CH = 128
TRASH = N

E_TOT = E + N
CPT_E = -(-E_TOT // (NS * CH))
E_PAD = CPT_E * NS * CH
CPT_D = -(-E // (NC * NS * CH))
E_DPAD = CPT_D * NC * NS * CH

@functools.lru_cache(maxsize=None)
def _sc_mesh():
    return plsc.VectorSubcoreMesh(
        core_axis_name="c", subcore_axis_name="s",
        num_cores=NC, num_subcores=NS)


def _deg_body(src_hbm, dst_hbm, perm_hbm, deg_out, q_out,
              perm_v, ones_v, tmp_v, srcb, dstb, qb, deg_sh):
    c = lax.axis_index("c")
    s = lax.axis_index("s")
    w = c * NS + s

    def _fill_ones(r, _):
        ones_v[r, :] = jnp.ones((L,), jnp.float32)
        tmp_v[r, :] = jnp.zeros((L,), jnp.float32)
        return 0
    lax.fori_loop(0, CH, _fill_ones, 0)

    def _zero_slice(m, _):
        pltpu.sync_copy(tmp_v, deg_sh.at[pl.ds(s * ROWS_PER_TILE + m * CH, CH)])
        return 0
    lax.fori_loop(0, ROWS_PER_TILE // CH, _zero_slice, 0)
    pltpu.sync_copy(perm_hbm, perm_v)
    plsc.subcore_barrier()

    base = w * CPT_D * CH

    def _chunk(j, _):
        eb = base + j * CH
        pltpu.sync_copy(src_hbm.at[pl.ds(eb, CH)], srcb)
        pltpu.sync_copy(dst_hbm.at[pl.ds(eb, CH)], dstb)
        pltpu.sync_copy(ones_v, deg_sh.at[dstb], add=True)

        def _qgather(k, _2):
            iv = srcb[pl.ds(k * L, L)]
            qb[pl.ds(k * L, L)] = plsc.load_gather(perm_v, [iv])
            return 0
        lax.fori_loop(0, CH // L, _qgather, 0)
        pltpu.sync_copy(qb, q_out.at[pl.ds(eb, CH)])
        return 0
    lax.fori_loop(0, CPT_D, _chunk, 0)
    plsc.subcore_barrier()

    def _wb(m, _):
        rb = s * ROWS_PER_TILE + m * CH
        pltpu.sync_copy(deg_sh.at[pl.ds(rb, CH)], tmp_v)
        pltpu.sync_copy(tmp_v, deg_out.at[c, pl.ds(rb, CH)])
        return 0
    lax.fori_loop(0, ROWS_PER_TILE // CH, _wb, 0)


@functools.lru_cache(maxsize=None)
def _deg_call():
  return pl.kernel(
    _deg_body,
    out_type=(
        jax.ShapeDtypeStruct((NC, NP, L), jnp.float32),
        jax.ShapeDtypeStruct((E_DPAD,), jnp.int32),
    ),
    mesh=_sc_mesh(),
    compiler_params=pltpu.CompilerParams(needs_layout_passes=False),
    scratch_types=[
        pltpu.VMEM((NP,), jnp.int32),
        pltpu.VMEM((CH, L), jnp.float32),
        pltpu.VMEM((CH, L), jnp.float32),
        pltpu.VMEM((CH,), jnp.int32),
        pltpu.VMEM((CH,), jnp.int32),
        pltpu.VMEM((CH,), jnp.int32),
        pltpu.VMEM_SHARED((NP, L), jnp.float32),
    ],
  )


def _conv_body(tt_hbm, g_hbm, dst_hbm, out_hbm,
               rows0, rows1, gb0, gb1, db0, db1,
               semg0, semg1, semd0, semd1, semr0, semr1, accum_sh):
    c = lax.axis_index("c")
    s = lax.axis_index("s")
    rows = (rows0, rows1)
    gb = (gb0, gb1)
    db = (db0, db1)
    semg = (semg0, semg1)
    semd = (semd0, semd1)
    semr = (semr0, semr1)

    def _zero_row(r, _):
        def _zc(k, _2):
            rows0[r, pl.ds(k * L, L)] = jnp.zeros((L,), jnp.float32)
            return 0
        lax.fori_loop(0, D // L, _zc, 0)
        return 0
    lax.fori_loop(0, CH, _zero_row, 0)

    def _zero_slice(m, _):
        pltpu.sync_copy(rows0, accum_sh.at[pl.ds(s * ROWS_PER_TILE + m * CH, CH)])
        return 0
    lax.fori_loop(0, ROWS_PER_TILE // CH, _zero_slice, 0)
    plsc.subcore_barrier()

    base = s * CPT_E * CH

    def _idx_start(j, b):
        eb = base + j * CH
        pltpu.async_copy(g_hbm.at[c, pl.ds(eb, CH)], gb[b], semg[b])
        pltpu.async_copy(dst_hbm.at[pl.ds(eb, CH)], db[b], semd[b])

    def _idx_wait(b):
        pltpu.make_async_copy(g_hbm.at[c, pl.ds(base, CH)], gb[b], semg[b]).wait()
        pltpu.make_async_copy(dst_hbm.at[pl.ds(base, CH)], db[b], semd[b]).wait()

    def _gather_start(b):
        pltpu.async_copy(tt_hbm.at[gb[b]], rows[b], semr[b])

    def _gather_wait(b):
        pltpu.make_async_copy(tt_hbm.at[gb[b]], rows[b], semr[b]).wait()

    _idx_start(0, 0)
    _idx_wait(0)
    _gather_start(0)
    _idx_start(1, 1)

    def _step(j, _):
        b = lax.rem(j, 2)

        def _one(b, nb):
            @pl.when(j + 1 < CPT_E)
            def _():
                _idx_wait(nb)
                _gather_start(nb)
            _gather_wait(b)
            pltpu.sync_copy(rows[b], accum_sh.at[db[b]], add=True)

            @pl.when(j + 2 < CPT_E)
            def _():
                _idx_start(j + 2, b)

        @pl.when(b == 0)
        def _():
            _one(0, 1)

        @pl.when(b == 1)
        def _():
            _one(1, 0)
        return 0
    lax.fori_loop(0, CPT_E, _step, 0)
    plsc.subcore_barrier()

    def _wb(m, _):
        rb = s * ROWS_PER_TILE + m * CH
        pltpu.sync_copy(accum_sh.at[pl.ds(rb, CH)], rows0)
        pltpu.sync_copy(rows0, out_hbm.at[c, pl.ds(rb, CH)])
        return 0
    lax.fori_loop(0, ROWS_PER_TILE // CH, _wb, 0)


@functools.lru_cache(maxsize=None)
def _conv_call():
  return pl.kernel(
    _conv_body,
    out_type=jax.ShapeDtypeStruct((NC, NP, D), jnp.float32),
    mesh=_sc_mesh(),
    compiler_params=pltpu.CompilerParams(needs_layout_passes=False),
    scratch_types=[
        pltpu.VMEM((CH, D), jnp.float32),
        pltpu.VMEM((CH, D), jnp.float32),
        pltpu.VMEM((CH,), jnp.int32),
        pltpu.VMEM((CH,), jnp.int32),
        pltpu.VMEM((CH,), jnp.int32),
        pltpu.VMEM((CH,), jnp.int32),
        pltpu.SemaphoreType.DMA,
        pltpu.SemaphoreType.DMA,
        pltpu.SemaphoreType.DMA,
        pltpu.SemaphoreType.DMA,
        pltpu.SemaphoreType.DMA,
        pltpu.SemaphoreType.DMA,
        pltpu.VMEM_SHARED((NP, D), jnp.float32),
    ],
  )


_BM = 512


def _tc1_body(x_ref, w_ref, dinv_ref, dperm_ref, out_ref):
    h = jnp.dot(x_ref[...], w_ref[...], preferred_element_type=jnp.float32)
    out_ref[0] = h * dinv_ref[...]
    out_ref[1] = h * dperm_ref[...]


def _tc1_call(x_pad, w1, dinv_c, dperm_c):
    grid = NP // _BM
    return pl.pallas_call(
        _tc1_body,
        grid=(grid,),
        in_specs=[
            pl.BlockSpec((_BM, D), lambda i: (i, 0)),
            pl.BlockSpec((D, D), lambda i: (0, 0)),
            pl.BlockSpec((_BM, 1), lambda i: (i, 0)),
            pl.BlockSpec((_BM, 1), lambda i: (i, 0)),
        ],
        out_specs=pl.BlockSpec((2, _BM, D), lambda i: (0, i, 0)),
        out_shape=jax.ShapeDtypeStruct((2, NP, D), jnp.float32),
    )(x_pad, w1, dinv_c, dperm_c)


def _tc2_body(a_ref, w_ref, dinv_ref, b_ref, alpha_ref, out_ref):
    dinv = dinv_ref[...]
    alpha = alpha_ref[0, 0]
    p = a_ref[0] * dinv + b_ref[...]
    p = jnp.where(p > 0, p, alpha * p)
    q = a_ref[1] * dinv + b_ref[...]
    q = jnp.where(q > 0, q, alpha * q)
    out_ref[0] = jnp.dot(p, w_ref[...], preferred_element_type=jnp.float32) * dinv
    out_ref[1] = jnp.dot(q, w_ref[...], preferred_element_type=jnp.float32) * dinv


def _tc2_call(a1_, w2, dinv_c, b1_, alpha_):
    grid = NP // _BM
    return pl.pallas_call(
        _tc2_body,
        grid=(grid,),
        in_specs=[
            pl.BlockSpec((2, _BM, D), lambda i: (0, i, 0)),
            pl.BlockSpec((D, D), lambda i: (0, 0)),
            pl.BlockSpec((_BM, 1), lambda i: (i, 0)),
            pl.BlockSpec((1, D), lambda i: (0, 0)),
            pl.BlockSpec((1, 1), lambda i: (0, 0)),
        ],
        out_specs=pl.BlockSpec((2, _BM, D), lambda i: (0, i, 0)),
        out_shape=jax.ShapeDtypeStruct((2, NP, D), jnp.float32),
    )(a1_, w2, dinv_c, b1_, alpha_)


_BM3 = 80
_NBLK3 = N // _BM3


def _tc3_body(a_ref, dinv_ref, b_ref, pos_ref, neg_ref, sum_ref):
    i = pl.program_id(0)
    dinv = dinv_ref[...]
    b = b_ref[...]
    p = a_ref[0] * dinv + b
    q = a_ref[1] * dinv + b
    pos_ref[...] = p
    neg_ref[...] = q
    colsum = jnp.sum(p, axis=0, keepdims=True)
    prev = jnp.where(i == 0, jnp.zeros_like(colsum), sum_ref[...])
    acc = prev + colsum
    sum_ref[...] = jnp.where(i == _NBLK3 - 1,
                             jax.nn.sigmoid(acc * (1.0 / N)), acc)


def _tc3_call(a2_, dinv_c, b2_):
    return pl.pallas_call(
        _tc3_body,
        grid=(_NBLK3,),
        in_specs=[
            pl.BlockSpec((2, _BM3, D), lambda i: (0, i, 0)),
            pl.BlockSpec((_BM3, 1), lambda i: (i, 0)),
            pl.BlockSpec((1, D), lambda i: (0, 0)),
        ],
        out_specs=[
            pl.BlockSpec((_BM3, D), lambda i: (i, 0)),
            pl.BlockSpec((_BM3, D), lambda i: (i, 0)),
            pl.BlockSpec((1, D), lambda i: (0, 0)),
        ],
        out_shape=[
            jax.ShapeDtypeStruct((N, D), jnp.float32),
            jax.ShapeDtypeStruct((N, D), jnp.float32),
            jax.ShapeDtypeStruct((1, D), jnp.float32),
        ],
    )(a2_, dinv_c, b2_)


def kernel(x, edge_index, W1, b1, a1, W2, b2):
    src = edge_index[0].astype(jnp.int32)
    dst = edge_index[1].astype(jnp.int32)
    perm = jax.random.permutation(jax.random.key(42), N).astype(jnp.int32)
    perm_inv = jnp.argsort(perm).astype(jnp.int32)
    iota = jnp.arange(N, dtype=jnp.int32)

    pad_d = E_DPAD - E
    srcp = jnp.concatenate([src, jnp.zeros((pad_d,), jnp.int32)])
    dstp = jnp.concatenate([dst, jnp.full((pad_d,), TRASH, jnp.int32)])
    perm_pad = jnp.concatenate([perm, jnp.zeros((NP - N,), jnp.int32)])
    deg2, q = _deg_call()(srcp, dstp, perm_pad)
    deg = deg2[0, :, 0] + deg2[1, :, 0] + 1.0
    dinv = lax.rsqrt(deg)
    dperm = jnp.concatenate([dinv[perm_inv], jnp.ones((NP - N,), jnp.float32)])
    dinv_c = dinv[:, None]
    dperm_c = dperm[:, None]

    x_pad = jnp.concatenate([x, jnp.zeros((NP - N, D), x.dtype)])
    tt1 = _tc1_call(x_pad, W1, dinv_c, dperm_c)

    pad_e = E_PAD - E_TOT
    zpad = jnp.zeros((pad_e,), jnp.int32)
    g_pos = jnp.concatenate([src, iota, zpad])
    g_neg = jnp.concatenate([q[:E], perm, zpad]) + NP
    g1 = jnp.stack([g_pos, g_neg])
    g2 = jnp.stack([g_pos, g_pos + NP])
    dste = jnp.concatenate([dst, iota, jnp.full((pad_e,), TRASH, jnp.int32)])

    a1_acc = _conv_call()(tt1.reshape(2 * NP, D), g1, dste)

    tt2 = _tc2_call(a1_acc, W2, dinv_c, b1.reshape(1, D), a1.reshape(1, 1))

    a2_acc = _conv_call()(tt2.reshape(2 * NP, D), g2, dste)

    pos, neg, summ = _tc3_call(a2_acc, dinv_c, b2.reshape(1, D))
    return (pos, neg, summ.reshape(D))
